# baseline (speedup 1.0000x reference)
import functools

import jax
import jax.numpy as jnp
from jax import lax
from jax.experimental import pallas as pl
from jax.experimental.pallas import tpu as pltpu
from jax.experimental.pallas import tpu_sc as plsc

_B = 16384
_D_DRV, _D_WK, _D_TM = 16, 3, 8
_D_OUT = _D_DRV + _D_WK + _D_TM + 1
_V_WK, _V_TM = 7, 1440
_V_DRV = 1000000

_BLK = 128
_RPB = _BLK // _D_DRV
_NBLK = _V_DRV // _RPB

_NC, _NS = 2, 16
_NW = _NC * _NS
_BPW = _B // _NW
_L = 16
_NCH = _BPW // _L
_GCH = 128
_NG = _BPW // _GCH


def _attr_body(drv_hbm, wk_hbm, tm_hbm, dist_hbm, wd_hbm, ww_hbm, wt_hbm,
               out_hbm, didx_v, bidx_v, widx_v, tidx_v, dist_v, blocks_v,
               wtab_v, ttab_v, out_v, sem):
  wid = lax.axis_index("s") * _NC + lax.axis_index("c")
  base = wid * _BPW

  pltpu.sync_copy(drv_hbm.at[pl.ds(base, _BPW)], didx_v)

  def mk_bidx(i, carry):
    d = didx_v[pl.ds(i * _L, _L)]
    bidx_v[pl.ds(i * _L, _L)] = d // _RPB
    return carry

  lax.fori_loop(0, _NCH, mk_bidx, 0)

  copies = [
      pltpu.async_copy(wd_hbm.at[bidx_v.at[pl.ds(k * _GCH, _GCH)]],
                       blocks_v.at[pl.ds(k * _GCH, _GCH)], sem)
      for k in range(_NG)
  ]

  pltpu.sync_copy(wk_hbm.at[pl.ds(base, _BPW)], widx_v)
  pltpu.sync_copy(tm_hbm.at[pl.ds(base, _BPW)], tidx_v)
  pltpu.sync_copy(dist_hbm.at[pl.ds(base, _BPW)], dist_v)
  pltpu.sync_copy(ww_hbm, wtab_v)
  pltpu.sync_copy(wt_hbm, ttab_v)

  def tail_chunk(ch, carry):
    rows = ch * _L + lax.iota(jnp.int32, _L)
    obase = rows * _D_OUT
    widx = widx_v[pl.ds(ch * _L, _L)] * _D_WK
    tidx = tidx_v[pl.ds(ch * _L, _L)] * _D_TM
    d = dist_v[pl.ds(ch * _L, _L)]
    for j in range(_D_WK):
      v = plsc.load_gather(wtab_v, [widx + j])
      plsc.store_scatter(out_v, [obase + (_D_DRV + j)], v)
    for j in range(_D_TM):
      v = plsc.load_gather(ttab_v, [tidx + j])
      plsc.store_scatter(out_v, [obase + (_D_DRV + _D_WK + j)], v)
    dn = ((d - 10.0) / 5.0 - 10.0) / 5.0
    plsc.store_scatter(out_v, [obase + (_D_OUT - 1)], dn)
    return carry

  lax.fori_loop(0, _NCH, tail_chunk, 0)

  for c in copies:
    c.wait()

  def drv_chunk(ch, carry):
    rows = ch * _L + lax.iota(jnp.int32, _L)
    obase = rows * _D_OUT
    sub = didx_v[pl.ds(ch * _L, _L)] % _RPB
    cbase = sub * _D_DRV
    for j in range(_D_DRV):
      v = plsc.load_gather(blocks_v, [rows, cbase + j])
      plsc.store_scatter(out_v, [obase + j], v)
    return carry

  lax.fori_loop(0, _NCH, drv_chunk, 0)

  pltpu.sync_copy(out_v, out_hbm.at[pl.ds(base * _D_OUT, _BPW * _D_OUT)])


def _build_kernel():
  return pl.kernel(
      _attr_body,
      out_type=jax.ShapeDtypeStruct((_B * _D_OUT,), jnp.float32),
      mesh=plsc.VectorSubcoreMesh(core_axis_name="c", subcore_axis_name="s"),
      compiler_params=pltpu.CompilerParams(needs_layout_passes=False),
      scratch_types=[
          pltpu.VMEM((_BPW,), jnp.int32),
          pltpu.VMEM((_BPW,), jnp.int32),
          pltpu.VMEM((_BPW,), jnp.int32),
          pltpu.VMEM((_BPW,), jnp.int32),
          pltpu.VMEM((_BPW,), jnp.float32),
          pltpu.VMEM((_BPW, _BLK), jnp.float32),
          pltpu.VMEM((_V_WK * _D_WK,), jnp.float32),
          pltpu.VMEM((_V_TM * _D_TM,), jnp.float32),
          pltpu.VMEM((_BPW * _D_OUT,), jnp.float32),
          pltpu.SemaphoreType.DMA,
      ],
  )


def kernel(driverID, weekID, timeID, dist, W_driver, W_week, W_time):
  drv = driverID.reshape(_B).astype(jnp.int32)
  wk = weekID.reshape(_B).astype(jnp.int32)
  tm = timeID.reshape(_B).astype(jnp.int32)
  d = dist.reshape(_B).astype(jnp.float32)
  wd = W_driver.reshape(_NBLK, _BLK)
  ww = W_week.reshape(_V_WK * _D_WK)
  wt = W_time.reshape(_V_TM * _D_TM)
  out = _build_kernel()(drv, wk, tm, d, wd, ww, wt)
  return out.reshape(_B, _D_OUT)
